# baseline (speedup 1.0000x reference)
import functools

import jax
import jax.numpy as jnp
from jax import lax
from jax.experimental import pallas as pl
from jax.experimental.pallas import tpu as pltpu
from jax.experimental.pallas import tpu_sc as plsc

NUM_EMB = 1_000_000
D = 32
B = 16384
L = 50

NC = 2
NS = 16
NW = NC * NS
BPW = B // NW
LANES = 16
KCH = BPW // LANES
NIDX = L * BPW
IDX_ROWS = NIDX // 128
BATCH_ROWS = 20
N_BATCH = IDX_ROWS // BATCH_ROWS

POS_PER_BLK = 131072
N_BLK = 8
TWN = N_BLK * POS_PER_BLK


def _matvec_body(w_ref, t_ref, o_ref):
    r1 = jnp.dot(w_ref[...], t_ref[...], preferred_element_type=jnp.float32)
    o_ref[...] = r1[0]


def _table_matvec(w_row, table_t):
    return pl.pallas_call(
        _matvec_body,
        grid=(N_BLK,),
        in_specs=[
            pl.BlockSpec((1, D), lambda i: (0, 0)),
            pl.BlockSpec((D, POS_PER_BLK), lambda i: (0, i)),
        ],
        out_specs=pl.BlockSpec((POS_PER_BLK,), lambda i: (i,)),
        out_shape=jax.ShapeDtypeStruct((TWN,), jnp.float32),
    )(w_row, table_t)


_MESH = plsc.VectorSubcoreMesh(core_axis_name="c", subcore_axis_name="s")


@functools.partial(
    pl.kernel,
    out_type=jax.ShapeDtypeStruct((B,), jnp.float32),
    mesh=_MESH,
    scratch_types=[
        pltpu.VMEM((IDX_ROWS, 128), jnp.int32),
        pltpu.VMEM((NIDX,), jnp.float32),
        pltpu.VMEM((BPW,), jnp.float32),
        pltpu.VMEM((LANES,), jnp.float32),
        pltpu.SemaphoreType.DMA,
        pltpu.SemaphoreType.DMA,
    ],
)
def _sc_gather_sum(xt_hbm, tw_hbm, bias_hbm, out_hbm,
                   idx_v, vals_v, acc_v, bias_v, sem0, sem1):
    wid = lax.axis_index("s") * NC + lax.axis_index("c")
    pltpu.sync_copy(xt_hbm.at[wid], idx_v)
    pltpu.sync_copy(bias_hbm, bias_v)

    bias_vec = bias_v[...]
    for k in range(KCH):
        acc_v[pl.ds(k * LANES, LANES)] = bias_vec

    def _fire_batch(b, sem):
        for r in range(BATCH_ROWS):
            i = b * BATCH_ROWS + r
            pltpu.async_copy(
                tw_hbm.at[idx_v.at[i]], vals_v.at[pl.ds(i * 128, 128)], sem)

    def _drain_accum_batch(b, sem):
        for r in range(BATCH_ROWS):
            i = b * BATCH_ROWS + r
            pltpu.make_async_copy(
                tw_hbm.at[idx_v.at[i]], vals_v.at[pl.ds(i * 128, 128)], sem
            ).wait()
        for r in range(BATCH_ROWS):
            i = b * BATCH_ROWS + r
            lane0 = (i & 3) * 128
            for k in range(8):
                chunk = vals_v[pl.ds(i * 128 + k * LANES, LANES)]
                plsc.addupdate(acc_v.at[pl.ds(lane0 + k * LANES, LANES)], chunk)

    _fire_batch(0, sem0)
    _fire_batch(1, sem1)

    @pl.loop(0, N_BATCH // 2 - 1)
    def _ring(t):
        b = t * 2
        _drain_accum_batch(b, sem0)
        _fire_batch(b + 2, sem0)
        _drain_accum_batch(b + 1, sem1)
        _fire_batch(b + 3, sem1)

    _drain_accum_batch(N_BATCH - 2, sem0)
    _drain_accum_batch(N_BATCH - 1, sem1)

    pltpu.sync_copy(acc_v, out_hbm.at[pl.ds(wid * BPW, BPW)])


def kernel(x, table, W, b):
    table_t = table.T
    w_row = (W.astype(jnp.float32) / L).T
    tw = _table_matvec(w_row, table_t)
    xt = x.reshape(NW, BPW, L).transpose(0, 2, 1).reshape(NW, IDX_ROWS, 128)
    bias16 = jnp.broadcast_to(b.astype(jnp.float32), (LANES,))
    out = _sc_gather_sum(xt, tw, bias16)
    return out.reshape(B, 1)


# speedup vs baseline: 1.0225x; 1.0225x over previous
import functools

import jax
import jax.numpy as jnp
from jax import lax
from jax.experimental import pallas as pl
from jax.experimental.pallas import tpu as pltpu
from jax.experimental.pallas import tpu_sc as plsc

NUM_EMB = 1_000_000
D = 32
B = 16384
L = 50

NC = 2
NS = 16
NW = NC * NS
BPW = B // NW
LANES = 16
KCH = BPW // LANES
NIDX = L * BPW
IDX_ROWS = NIDX // 128
BATCH_ROWS = 20
N_BATCH = IDX_ROWS // BATCH_ROWS

POS_PER_BLK = 65536
N_BLK = 16
TWN = N_BLK * POS_PER_BLK


def _matvec_body(w_ref, t_ref, o_ref):
    r1 = jnp.dot(w_ref[...], t_ref[...], preferred_element_type=jnp.float32)
    o_ref[...] = r1[0]


def _table_matvec(w_row, table_t):
    return pl.pallas_call(
        _matvec_body,
        grid=(N_BLK,),
        in_specs=[
            pl.BlockSpec((1, D), lambda i: (0, 0)),
            pl.BlockSpec((D, POS_PER_BLK), lambda i: (0, i)),
        ],
        out_specs=pl.BlockSpec((POS_PER_BLK,), lambda i: (i,)),
        out_shape=jax.ShapeDtypeStruct((TWN,), jnp.float32),
    )(w_row, table_t)


_MESH = plsc.VectorSubcoreMesh(core_axis_name="c", subcore_axis_name="s")


@functools.partial(
    pl.kernel,
    out_type=jax.ShapeDtypeStruct((B,), jnp.float32),
    mesh=_MESH,
    scratch_types=[
        pltpu.VMEM((IDX_ROWS, 128), jnp.int32),
        pltpu.VMEM((NIDX,), jnp.float32),
        pltpu.VMEM((BPW,), jnp.float32),
        pltpu.VMEM((LANES,), jnp.float32),
        pltpu.SemaphoreType.DMA,
        pltpu.SemaphoreType.DMA,
    ],
)
def _sc_gather_sum(xt_hbm, tw_hbm, bias_hbm, out_hbm,
                   idx_v, vals_v, acc_v, bias_v, sem0, sem1):
    wid = lax.axis_index("s") * NC + lax.axis_index("c")
    pltpu.sync_copy(xt_hbm.at[wid], idx_v)
    pltpu.sync_copy(bias_hbm, bias_v)

    bias_vec = bias_v[...]
    for k in range(KCH):
        acc_v[pl.ds(k * LANES, LANES)] = bias_vec

    def _fire_batch(b, sem):
        for r in range(BATCH_ROWS):
            i = b * BATCH_ROWS + r
            pltpu.async_copy(
                tw_hbm.at[idx_v.at[i]], vals_v.at[pl.ds(i * 128, 128)], sem)

    def _drain_accum_batch(b, sem):
        for r in range(BATCH_ROWS):
            i = b * BATCH_ROWS + r
            pltpu.make_async_copy(
                tw_hbm.at[idx_v.at[i]], vals_v.at[pl.ds(i * 128, 128)], sem
            ).wait()
        for r in range(BATCH_ROWS):
            i = b * BATCH_ROWS + r
            lane0 = (i & 3) * 128
            for k in range(8):
                chunk = vals_v[pl.ds(i * 128 + k * LANES, LANES)]
                plsc.addupdate(acc_v.at[pl.ds(lane0 + k * LANES, LANES)], chunk)

    _fire_batch(0, sem0)
    _fire_batch(1, sem1)

    @pl.loop(0, N_BATCH // 2 - 1)
    def _ring(t):
        b = t * 2
        _drain_accum_batch(b, sem0)
        _fire_batch(b + 2, sem0)
        _drain_accum_batch(b + 1, sem1)
        _fire_batch(b + 3, sem1)

    _drain_accum_batch(N_BATCH - 2, sem0)
    _drain_accum_batch(N_BATCH - 1, sem1)

    pltpu.sync_copy(acc_v, out_hbm.at[pl.ds(wid * BPW, BPW)])


def kernel(x, table, W, b):
    table_t = table.T
    w_row = (W.astype(jnp.float32) / L).T
    tw = _table_matvec(w_row, table_t)
    xt = x.reshape(NW, BPW, L).transpose(0, 2, 1).reshape(NW, IDX_ROWS, 128)
    bias16 = jnp.broadcast_to(b.astype(jnp.float32), (LANES,))
    out = _sc_gather_sum(xt, tw, bias16)
    return out.reshape(B, 1)


# speedup vs baseline: 1.0384x; 1.0156x over previous
import functools

import jax
import jax.numpy as jnp
from jax import lax
from jax.experimental import pallas as pl
from jax.experimental.pallas import tpu as pltpu
from jax.experimental.pallas import tpu_sc as plsc

NUM_EMB = 1_000_000
D = 32
B = 16384
L = 50

NC = 2
NS = 16
NW = NC * NS
BPW = B // NW
LANES = 16
KCH = BPW // LANES
NIDX = L * BPW
IDX_ROWS = NIDX // 128
BATCH_ROWS = 10
N_BATCH = IDX_ROWS // BATCH_ROWS

POS_PER_BLK = 65536
N_BLK = 16
TWN = N_BLK * POS_PER_BLK


def _matvec_body(w_ref, t_ref, o_ref):
    r1 = jnp.dot(w_ref[...], t_ref[...], preferred_element_type=jnp.float32)
    o_ref[...] = r1[0]


def _table_matvec(w_row, table_t):
    return pl.pallas_call(
        _matvec_body,
        grid=(N_BLK,),
        in_specs=[
            pl.BlockSpec((1, D), lambda i: (0, 0)),
            pl.BlockSpec((D, POS_PER_BLK), lambda i: (0, i)),
        ],
        out_specs=pl.BlockSpec((POS_PER_BLK,), lambda i: (i,)),
        out_shape=jax.ShapeDtypeStruct((TWN,), jnp.float32),
    )(w_row, table_t)


_MESH = plsc.VectorSubcoreMesh(core_axis_name="c", subcore_axis_name="s")


@functools.partial(
    pl.kernel,
    out_type=jax.ShapeDtypeStruct((B,), jnp.float32),
    mesh=_MESH,
    scratch_types=[
        pltpu.VMEM((IDX_ROWS, 128), jnp.int32),
        pltpu.VMEM((NIDX,), jnp.float32),
        pltpu.VMEM((BPW,), jnp.float32),
        pltpu.VMEM((LANES,), jnp.float32),
        pltpu.SemaphoreType.DMA,
        pltpu.SemaphoreType.DMA,
        pltpu.SemaphoreType.DMA,
        pltpu.SemaphoreType.DMA,
    ],
)
def _sc_gather_sum(xt_hbm, tw_hbm, bias_hbm, out_hbm,
                   idx_v, vals_v, acc_v, bias_v, sem0, sem1, sem2, sem3):
    wid = lax.axis_index("s") * NC + lax.axis_index("c")
    pltpu.sync_copy(xt_hbm.at[wid], idx_v)
    pltpu.sync_copy(bias_hbm, bias_v)

    bias_vec = bias_v[...]
    for k in range(KCH):
        acc_v[pl.ds(k * LANES, LANES)] = bias_vec

    def _fire_batch(b, sem):
        for r in range(BATCH_ROWS):
            i = b * BATCH_ROWS + r
            pltpu.async_copy(
                tw_hbm.at[idx_v.at[i]], vals_v.at[pl.ds(i * 128, 128)], sem)

    def _drain_accum_batch(b, sem):
        for r in range(BATCH_ROWS):
            i = b * BATCH_ROWS + r
            pltpu.make_async_copy(
                tw_hbm.at[idx_v.at[i]], vals_v.at[pl.ds(i * 128, 128)], sem
            ).wait()
        for r in range(BATCH_ROWS):
            i = b * BATCH_ROWS + r
            lane0 = (i & 3) * 128
            for k in range(8):
                chunk = vals_v[pl.ds(i * 128 + k * LANES, LANES)]
                plsc.addupdate(acc_v.at[pl.ds(lane0 + k * LANES, LANES)], chunk)

    sems = (sem0, sem1, sem2, sem3)
    nsem = len(sems)
    for b0 in range(nsem):
        _fire_batch(b0, sems[b0])

    @pl.loop(0, N_BATCH // nsem - 1)
    def _ring(t):
        b = t * nsem
        for s in range(nsem):
            _drain_accum_batch(b + s, sems[s])
            _fire_batch(b + s + nsem, sems[s])

    for s in range(nsem):
        _drain_accum_batch(N_BATCH - nsem + s, sems[s])

    pltpu.sync_copy(acc_v, out_hbm.at[pl.ds(wid * BPW, BPW)])


def kernel(x, table, W, b):
    table_t = table.T
    w_row = (W.astype(jnp.float32) / L).T
    tw = _table_matvec(w_row, table_t)
    xt = x.reshape(NW, BPW, L).transpose(0, 2, 1).reshape(NW, IDX_ROWS, 128)
    bias16 = jnp.broadcast_to(b.astype(jnp.float32), (LANES,))
    out = _sc_gather_sum(xt, tw, bias16)
    return out.reshape(B, 1)


# speedup vs baseline: 1.0653x; 1.0259x over previous
import functools

import jax
import jax.numpy as jnp
from jax import lax
from jax.experimental import pallas as pl
from jax.experimental.pallas import tpu as pltpu
from jax.experimental.pallas import tpu_sc as plsc

NUM_EMB = 1_000_000
D = 32
B = 16384
L = 50

NC = 2
NS = 16
NW = NC * NS
BPW = B // NW
LANES = 16
KCH = BPW // LANES
NIDX = L * BPW
IDX_ROWS = NIDX // 128
BATCH_ROWS = 10
N_BATCH = IDX_ROWS // BATCH_ROWS

POS_PER_BLK = 65536
N_BLK = 16
TWN = N_BLK * POS_PER_BLK


def _matvec_body(w_ref, t_ref, o_ref):
    r1 = jnp.dot(w_ref[...], t_ref[...], preferred_element_type=jnp.float32)
    o_ref[...] = r1[0]


def _table_matvec(w_row, table_t):
    return pl.pallas_call(
        _matvec_body,
        grid=(N_BLK,),
        in_specs=[
            pl.BlockSpec((1, D), lambda i: (0, 0)),
            pl.BlockSpec((D, POS_PER_BLK), lambda i: (0, i)),
        ],
        out_specs=pl.BlockSpec((POS_PER_BLK,), lambda i: (i,)),
        out_shape=jax.ShapeDtypeStruct((TWN,), jnp.float32),
    )(w_row, table_t)


_MESH = plsc.VectorSubcoreMesh(core_axis_name="c", subcore_axis_name="s")


@functools.partial(
    pl.kernel,
    out_type=jax.ShapeDtypeStruct((B,), jnp.float32),
    mesh=_MESH,
    scratch_types=[
        pltpu.VMEM((L, BPW), jnp.int32),
        pltpu.VMEM((NIDX,), jnp.float32),
        pltpu.VMEM((BPW,), jnp.float32),
        pltpu.VMEM((LANES,), jnp.float32),
        pltpu.SemaphoreType.DMA,
        pltpu.SemaphoreType.DMA,
    ],
)
def _sc_gather_sum(xt_hbm, tw_hbm, bias_hbm, out_hbm,
                   idx_v, vals_v, acc_v, bias_v, sem0, sem1):
    wid = lax.axis_index("s") * NC + lax.axis_index("c")
    pltpu.sync_copy(xt_hbm.at[:, wid], idx_v)
    pltpu.sync_copy(bias_hbm, bias_v)

    def _idx_row(i):
        return idx_v.at[i >> 2, pl.ds((i & 3) * 128, 128)]

    bias_vec = bias_v[...]
    for k in range(KCH):
        acc_v[pl.ds(k * LANES, LANES)] = bias_vec

    def _fire_batch(b, sem):
        for r in range(BATCH_ROWS):
            i = b * BATCH_ROWS + r
            pltpu.async_copy(
                tw_hbm.at[_idx_row(i)], vals_v.at[pl.ds(i * 128, 128)], sem)

    def _drain_accum_batch(b, sem):
        for r in range(BATCH_ROWS):
            i = b * BATCH_ROWS + r
            pltpu.make_async_copy(
                tw_hbm.at[_idx_row(i)], vals_v.at[pl.ds(i * 128, 128)], sem
            ).wait()
        for r in range(BATCH_ROWS):
            i = b * BATCH_ROWS + r
            lane0 = (i & 3) * 128
            for k in range(8):
                chunk = vals_v[pl.ds(i * 128 + k * LANES, LANES)]
                plsc.addupdate(acc_v.at[pl.ds(lane0 + k * LANES, LANES)], chunk)

    _fire_batch(0, sem0)
    _fire_batch(1, sem1)

    @pl.loop(0, N_BATCH // 2 - 1)
    def _ring(t):
        b = t * 2
        _drain_accum_batch(b, sem0)
        _fire_batch(b + 2, sem0)
        _drain_accum_batch(b + 1, sem1)
        _fire_batch(b + 3, sem1)

    _drain_accum_batch(N_BATCH - 2, sem0)
    _drain_accum_batch(N_BATCH - 1, sem1)

    pltpu.sync_copy(acc_v, out_hbm.at[pl.ds(wid * BPW, BPW)])


def kernel(x, table, W, b):
    table_t = table.T
    w_row = (W.astype(jnp.float32) / L).T
    tw = _table_matvec(w_row, table_t)
    xt = x.T.reshape(L, NW, BPW)
    bias16 = jnp.broadcast_to(b.astype(jnp.float32), (LANES,))
    out = _sc_gather_sum(xt, tw, bias16)
    return out.reshape(B, 1)


# speedup vs baseline: 1.1381x; 1.0684x over previous
import functools

import jax
import jax.numpy as jnp
from jax import lax
from jax.experimental import pallas as pl
from jax.experimental.pallas import tpu as pltpu
from jax.experimental.pallas import tpu_sc as plsc

NUM_EMB = 1_000_000
D = 32
B = 16384
L = 50

NC = 2
NS = 16
NW = NC * NS
BPW = B // NW
LANES = 16
KCH = BPW // LANES
NIDX = L * BPW
IDX_ROWS = NIDX // 128
BATCH_ROWS = 10
N_BATCH = IDX_ROWS // BATCH_ROWS

POS_PER_BLK = 65536
N_BLK = 16
TWN = N_BLK * POS_PER_BLK


def _matvec_body(w_ref, t_ref, o_ref):
    r1 = jnp.dot(w_ref[...], t_ref[...], preferred_element_type=jnp.float32)
    o_ref[...] = r1[0]


def _table_matvec(w_row, table_t):
    return pl.pallas_call(
        _matvec_body,
        grid=(N_BLK,),
        in_specs=[
            pl.BlockSpec((1, D), lambda i: (0, 0)),
            pl.BlockSpec((D, POS_PER_BLK), lambda i: (0, i)),
        ],
        out_specs=pl.BlockSpec((POS_PER_BLK,), lambda i: (i,)),
        out_shape=jax.ShapeDtypeStruct((TWN,), jnp.float32),
    )(w_row, table_t)


_MESH = plsc.VectorSubcoreMesh(core_axis_name="c", subcore_axis_name="s")


@functools.partial(
    pl.kernel,
    out_type=jax.ShapeDtypeStruct((B,), jnp.float32),
    mesh=_MESH,
    scratch_types=[
        pltpu.VMEM((L, BPW), jnp.int32),
        pltpu.VMEM((NIDX,), jnp.float32),
        pltpu.VMEM((BPW,), jnp.float32),
        pltpu.VMEM((LANES,), jnp.float32),
        pltpu.SemaphoreType.DMA,
        pltpu.SemaphoreType.DMA,
    ],
)
def _sc_gather_sum(xt_hbm, tw_hbm, bias_hbm, out_hbm,
                   idx_v, vals_v, acc_v, bias_v, sem0, sem1):
    wid = lax.axis_index("s") * NC + lax.axis_index("c")
    pltpu.sync_copy(xt_hbm.at[:, pl.ds(wid * BPW, BPW)], idx_v)
    pltpu.sync_copy(bias_hbm, bias_v)

    def _idx_row(i):
        return idx_v.at[i >> 2, pl.ds((i & 3) * 128, 128)]

    bias_vec = bias_v[...]
    for k in range(KCH):
        acc_v[pl.ds(k * LANES, LANES)] = bias_vec

    def _fire_batch(b, sem):
        for r in range(BATCH_ROWS):
            i = b * BATCH_ROWS + r
            pltpu.async_copy(
                tw_hbm.at[_idx_row(i)], vals_v.at[pl.ds(i * 128, 128)], sem)

    def _drain_accum_batch(b, sem):
        for r in range(BATCH_ROWS):
            i = b * BATCH_ROWS + r
            pltpu.make_async_copy(
                tw_hbm.at[_idx_row(i)], vals_v.at[pl.ds(i * 128, 128)], sem
            ).wait()
        for r in range(BATCH_ROWS):
            i = b * BATCH_ROWS + r
            lane0 = (i & 3) * 128
            for k in range(8):
                chunk = vals_v[pl.ds(i * 128 + k * LANES, LANES)]
                plsc.addupdate(acc_v.at[pl.ds(lane0 + k * LANES, LANES)], chunk)

    _fire_batch(0, sem0)
    _fire_batch(1, sem1)

    @pl.loop(0, N_BATCH // 2 - 1)
    def _ring(t):
        b = t * 2
        _drain_accum_batch(b, sem0)
        _fire_batch(b + 2, sem0)
        _drain_accum_batch(b + 1, sem1)
        _fire_batch(b + 3, sem1)

    _drain_accum_batch(N_BATCH - 2, sem0)
    _drain_accum_batch(N_BATCH - 1, sem1)

    pltpu.sync_copy(acc_v, out_hbm.at[pl.ds(wid * BPW, BPW)])


def kernel(x, table, W, b):
    table_t = table.T
    w_row = (W.astype(jnp.float32) / L).T
    tw = _table_matvec(w_row, table_t)
    xt = x.T
    bias16 = jnp.broadcast_to(b.astype(jnp.float32), (LANES,))
    out = _sc_gather_sum(xt, tw, bias16)
    return out.reshape(B, 1)
